# baseline (speedup 1.0000x reference)
import functools

import jax
import jax.numpy as jnp
from jax import lax
from jax.experimental import pallas as pl
from jax.experimental.pallas import tpu as pltpu
from jax.experimental.pallas import tpu_sc as plsc

_BATCH = 16384
_EMB = 16
_LEN_CONT = 8
_NTAB = 5


def _sc_gather(tables, idx_chunks, bpw, nw):
    n = _NTAB
    rows_per_w = n * bpw

    mesh = plsc.VectorSubcoreMesh(core_axis_name="c", subcore_axis_name="s")
    scratch = [
        pltpu.VMEM((rows_per_w,), jnp.int32),
        pltpu.VMEM((rows_per_w, _EMB), jnp.float32),
        pltpu.SemaphoreType.DMA,
    ]

    @functools.partial(
        pl.kernel,
        mesh=mesh,
        out_type=jax.ShapeDtypeStruct((nw * rows_per_w, _EMB), jnp.float32),
        scratch_types=scratch,
        compiler_params=pltpu.CompilerParams(use_tc_tiling_on_sc=False),
    )
    def k(*refs):
        tabs = refs[:n]
        idxs = refs[n]
        out = refs[n + 1]
        idx_v = refs[n + 2]
        rows_v = refs[n + 3]
        sem = refs[n + 4]
        nc = 2
        wid = lax.axis_index("s") * nc + lax.axis_index("c")
        pltpu.sync_copy(idxs.at[wid], idx_v)
        copies = [
            pltpu.async_copy(
                tabs[j].at[idx_v.at[pl.ds(j * bpw, bpw)]],
                rows_v.at[pl.ds(j * bpw, bpw)],
                sem,
            )
            for j in range(n)
        ]
        for c in copies:
            c.wait()
        pltpu.sync_copy(rows_v, out.at[pl.ds(wid * rows_per_w, rows_per_w)])

    return k(*tables, idx_chunks)


def _mlp(packed, cont, W1, b1, W2, b2, W3, b3, Wf, bf, bpw, nw):
    grid = (nw,)

    def body(e0, e1, e2, e3, e4, c, w1, v1, w2, v2, w3, v3, wf, vf, out):
        x = jnp.concatenate(
            [e0[...], e1[...], e2[...], e3[...], e4[...], c[...]], axis=1
        )
        h = jnp.maximum(
            jnp.dot(x, w1[...], preferred_element_type=jnp.float32) + v1[...], 0.0
        )
        h = jnp.maximum(
            jnp.dot(h, w2[...], preferred_element_type=jnp.float32) + v2[...], 0.0
        )
        h = jnp.maximum(
            jnp.dot(h, w3[...], preferred_element_type=jnp.float32) + v3[...], 0.0
        )
        logit = jnp.dot(h, wf[...], preferred_element_type=jnp.float32) + vf[...]
        out[...] = jax.nn.sigmoid(logit)

    def espec(j):
        return pl.BlockSpec((bpw, _EMB), lambda i, j=j: (i * _NTAB + j, 0))

    cb = pl.BlockSpec((bpw, _LEN_CONT), lambda i: (i, 0))

    def wspec(shape):
        return pl.BlockSpec(shape, lambda i: (0, 0))

    return pl.pallas_call(
        body,
        grid=grid,
        in_specs=[espec(j) for j in range(_NTAB)]
        + [cb]
        + [
            wspec((88, 64)),
            wspec((1, 64)),
            wspec((64, 32)),
            wspec((1, 32)),
            wspec((32, 16)),
            wspec((1, 16)),
            wspec((16, 1)),
            wspec((1, 1)),
        ],
        out_specs=pl.BlockSpec((bpw, 1), lambda i: (i, 0)),
        out_shape=jax.ShapeDtypeStruct((_BATCH, 1), jnp.float32),
    )(*([packed] * _NTAB), cont, W1, b1, W2, b2, W3, b3, Wf, bf)


def kernel(X_deep, session_table, promotion_table, age_table, gender_table,
           purchase_table, W1, b1, W2, b2, W3, b3, Wf, bf):
    nw = 32
    bpw = _BATCH // nw
    idx_chunks = (
        X_deep[:, :_NTAB].T.reshape(_NTAB, nw, bpw)
        .transpose(1, 0, 2)
        .reshape(nw, _NTAB * bpw)
    )
    cont = X_deep[:, _NTAB:].astype(jnp.float32)
    packed = _sc_gather(
        (session_table, promotion_table, age_table, gender_table, purchase_table),
        idx_chunks, bpw, nw,
    )
    return _mlp(
        packed, cont,
        W1, b1.reshape(1, 64),
        W2, b2.reshape(1, 32),
        W3, b3.reshape(1, 16),
        Wf, bf.reshape(1, 1),
        bpw, nw,
    )


# speedup vs baseline: 1.6982x; 1.6982x over previous
import functools

import jax
import jax.numpy as jnp
from jax import lax
from jax.experimental import pallas as pl
from jax.experimental.pallas import tpu as pltpu
from jax.experimental.pallas import tpu_sc as plsc

_BATCH = 16384
_EMB = 16
_LEN_CONT = 8
_NBIG = 100000
_NSHARED = 100048
_NS = 16
_NC = 2
_BPT = _BATCH // _NS
_HPT = _BATCH // (2 * _NS)
_STAGE = _NBIG // _NS
_NIDX = _BPT + 3 * _HPT
_CH = 512
_NCH = _NIDX // _CH


def _sc_gather(session_t, promotion_t, age_t, gender_t, purchase_t, idx_chunks):
    mesh = plsc.VectorSubcoreMesh(core_axis_name="c", subcore_axis_name="s")

    @functools.partial(
        pl.kernel,
        mesh=mesh,
        out_type=jax.ShapeDtypeStruct((_NC * _NS * _NIDX, _EMB), jnp.float32),
        scratch_types=[
            pltpu.VMEM_SHARED((_NSHARED, _EMB), jnp.float32),
            pltpu.VMEM((_NIDX,), jnp.int32),
            pltpu.VMEM((_CH, _EMB), jnp.float32),
            pltpu.VMEM((_CH, _EMB), jnp.float32),
            pltpu.SemaphoreType.DMA,
            pltpu.SemaphoreType.DMA,
        ],
        compiler_params=pltpu.CompilerParams(use_tc_tiling_on_sc=False),
    )
    def k(sess, promo, age, gen, pur, idxs, out,
          shared, idx_v, buf0, buf1, gsem, wsem):
        c = lax.axis_index("c")
        s = lax.axis_index("s")
        wid = c * _NS + s

        @pl.when(c == 0)
        def _():
            pltpu.sync_copy(sess.at[pl.ds(s * _STAGE, _STAGE)],
                            shared.at[pl.ds(s * _STAGE, _STAGE)])

        @pl.when(c == 1)
        def _():
            pltpu.sync_copy(promo.at[pl.ds(s * _STAGE, _STAGE)],
                            shared.at[pl.ds(s * _STAGE, _STAGE)])

        @pl.when(s == 0)
        def _():
            pltpu.sync_copy(age, shared.at[pl.ds(_NBIG, 20)])
            pltpu.sync_copy(gen, shared.at[pl.ds(_NBIG + 20, 12)])
            pltpu.sync_copy(pur, shared.at[pl.ds(_NBIG + 32, 10)])

        pltpu.sync_copy(idxs.at[wid], idx_v)
        plsc.subcore_barrier()

        bufs = (buf0, buf1)
        base = wid * _NIDX

        def gather(kk):
            return pltpu.async_copy(
                shared.at[idx_v.at[pl.ds(kk * _CH, _CH)]], bufs[kk % 2], gsem)

        def write(kk):
            return pltpu.async_copy(
                bufs[kk % 2], out.at[pl.ds(base + kk * _CH, _CH)], wsem)

        g_prev = gather(0)
        g_cur = gather(1)
        w_prev = None
        for kk in range(_NCH):
            g_prev.wait()
            if w_prev is not None:
                w_prev.wait()
            w_prev = write(kk)
            if kk + 2 < _NCH:
                g_next = gather(kk + 2)
            g_prev = g_cur
            g_cur = g_next if kk + 2 < _NCH else None
        w_prev.wait()

    return k(session_t, promotion_t, age_t, gender_t, purchase_t, idx_chunks)


def _mlp(packed, cont, W1, b1, W2, b2, W3, b3, Wf, bf):
    blk = 512
    grid = (_BATCH // blk,)

    def body(e0, e1, e2, e3, e4, cf, w1, v1, w2, v2, w3, v3, wf, vf, out):
        x = jnp.concatenate(
            [e0[...], e1[...], e2[...], e3[...], e4[...], cf[...]], axis=1
        )
        h = jnp.maximum(
            jnp.dot(x, w1[...], preferred_element_type=jnp.float32) + v1[...], 0.0
        )
        h = jnp.maximum(
            jnp.dot(h, w2[...], preferred_element_type=jnp.float32) + v2[...], 0.0
        )
        h = jnp.maximum(
            jnp.dot(h, w3[...], preferred_element_type=jnp.float32) + v3[...], 0.0
        )
        logit = jnp.dot(h, wf[...], preferred_element_type=jnp.float32) + vf[...]
        out[...] = jax.nn.sigmoid(logit)

    sess_spec = pl.BlockSpec((blk, _EMB), lambda i: (5 * (i // 2) + i % 2, 0))
    promo_spec = pl.BlockSpec(
        (blk, _EMB), lambda i: (5 * (_NS + i // 2) + i % 2, 0))

    def sm_spec(j):
        return pl.BlockSpec((blk, _EMB), lambda i, j=j: (5 * i + 2 + j, 0))

    cb = pl.BlockSpec((blk, _LEN_CONT), lambda i: (i, 0))

    def wspec(shape):
        return pl.BlockSpec(shape, lambda i: (0, 0))

    return pl.pallas_call(
        body,
        grid=grid,
        in_specs=[sess_spec, promo_spec, sm_spec(0), sm_spec(1), sm_spec(2)]
        + [cb]
        + [
            wspec((88, 64)),
            wspec((1, 64)),
            wspec((64, 32)),
            wspec((1, 32)),
            wspec((32, 16)),
            wspec((1, 16)),
            wspec((16, 1)),
            wspec((1, 1)),
        ],
        out_specs=pl.BlockSpec((blk, 1), lambda i: (i, 0)),
        out_shape=jax.ShapeDtypeStruct((_BATCH, 1), jnp.float32),
    )(*([packed] * 5), cont, W1, b1, W2, b2, W3, b3, Wf, bf)


def kernel(X_deep, session_table, promotion_table, age_table, gender_table,
           purchase_table, W1, b1, W2, b2, W3, b3, Wf, bf):
    big = X_deep[:, :2].T.reshape(2, _NS, _BPT)
    sm = (X_deep[:, 2:5]
          + jnp.array([_NBIG, _NBIG + 20, _NBIG + 32], jnp.int32)).T
    sm = sm.reshape(3, 2, _NS, _HPT).transpose(1, 2, 0, 3).reshape(
        2, _NS, 3 * _HPT)
    idx_chunks = jnp.concatenate([big, sm], axis=-1).reshape(2 * _NS, _NIDX)

    cont = X_deep[:, 5:].astype(jnp.float32)
    packed = _sc_gather(
        session_table, promotion_table, age_table, gender_table,
        purchase_table, idx_chunks,
    )
    return _mlp(
        packed, cont,
        W1, b1.reshape(1, 64),
        W2, b2.reshape(1, 32),
        W3, b3.reshape(1, 16),
        Wf, bf.reshape(1, 1),
    )
